# baseline (speedup 1.0000x reference)
import functools

import jax
import jax.numpy as jnp
from jax import lax
from jax.experimental import pallas as pl
from jax.experimental.pallas import tpu as pltpu
from jax.experimental.pallas import tpu_sc as plsc

K = 20
ROWS = 256
R3 = 256
GW = 128


def _knn_body(xfull_ref, xtile_ref, w1at_ref, wdt_ref, idx_ref, a_ref, bv_ref):
    xb = xfull_ref[0]
    xi = xtile_ref[0]
    n = xb.shape[1]
    rows = xi.shape[1]

    dn = (((0,), (0,)), ((), ()))
    a_ref[0] = lax.dot_general(xi, w1at_ref[...], dn,
                               preferred_element_type=jnp.float32)
    bv_ref[0] = lax.dot_general(xi, wdt_ref[...], dn,
                                preferred_element_type=jnp.float32)

    inner = lax.dot_general(xi, xb, dn, preferred_element_type=jnp.float32)
    xsq = jnp.sum(xb * xb, axis=0)
    xsq_i = jnp.sum(xi * xi, axis=0)
    d = 2.0 * inner - xsq[None, :] - xsq_i[:, None]

    nbits = (n - 1).bit_length()
    lowmask = jnp.int32((1 << nbits) - 1)
    rowmin = jnp.min(d, axis=1, keepdims=True)
    scale = (2.0 ** 20 - 2.0) / jnp.maximum(-rowmin, 1e-30)
    qd = lax.convert_element_type(d * scale, jnp.int32)
    cols = lax.broadcasted_iota(jnp.int32, (rows, n), 1)
    key = (qd * jnp.int32(1 << nbits)) | (jnp.int32(n - 1) - cols)

    neg_inf = jnp.int32(-(2 ** 31))
    base = pl.program_id(0) * n
    picks = []
    for _ in range(K):
        m = jnp.max(key, axis=1, keepdims=True)
        picks.append(jnp.int32(n - 1) - (m & lowmask) + base)
        key = jnp.where(key == m, neg_inf, key)
    idx_ref[0] = jnp.concatenate(picks, axis=1)


def _mlp_body(g_ref, bv_ref, w2t_ref, out_ref):
    h = g_ref[0] + bv_ref[...]
    h = jnp.where(h > 0, h, 0.2 * h)
    h2 = jnp.dot(h, w2t_ref[...], preferred_element_type=jnp.float32)
    h2 = jnp.where(h2 > 0, h2, 0.2 * h2)

    @pl.when(pl.program_id(1) == 0)
    def _():
        out_ref[...] = h2

    @pl.when(pl.program_id(1) != 0)
    def _():
        out_ref[...] = jnp.maximum(out_ref[...], h2)


def _sc_gather(table, idx):
    num = idx.shape[0]
    d_dim = table.shape[1]
    idx2 = idx.reshape(1, num)
    mesh = plsc.VectorSubcoreMesh(core_axis_name="c", subcore_axis_name="s")

    @functools.partial(
        pl.kernel,
        out_type=jax.ShapeDtypeStruct((num, d_dim), table.dtype),
        mesh=mesh,
    )
    def gk(table_hbm, idx_hbm, out_hbm):
        def body(i_vmem, o_vmem):
            pltpu.sync_copy(table_hbm.at[i_vmem.at[0]], o_vmem)

        pltpu.emit_pipeline(
            body,
            grid=(num // GW,),
            in_specs=[pl.BlockSpec((1, GW), lambda i: (0, i))],
            out_specs=[pl.BlockSpec((GW, d_dim), lambda i: (i, 0))],
            core_axis_name=("c", "s"),
            dimension_semantics=(pltpu.PARALLEL,),
        )(idx_hbm, out_hbm)

    return gk(table, idx2)


def kernel(x, W1, W2):
    B, C, N = x.shape
    O1 = W1.shape[0]
    O2 = W2.shape[0]
    w1at = jnp.transpose(W1[:, :C])
    wdt = jnp.transpose(W1[:, C:] - W1[:, :C])
    w2t = jnp.transpose(W2)

    idxg, a_rows, b_rows = pl.pallas_call(
        _knn_body,
        grid=(B, N // ROWS),
        in_specs=[
            pl.BlockSpec((1, C, N), lambda b, i: (b, 0, 0)),
            pl.BlockSpec((1, C, ROWS), lambda b, i: (b, 0, i)),
            pl.BlockSpec((C, O1), lambda b, i: (0, 0)),
            pl.BlockSpec((C, O1), lambda b, i: (0, 0)),
        ],
        out_specs=[
            pl.BlockSpec((1, ROWS, K), lambda b, i: (b, i, 0)),
            pl.BlockSpec((1, ROWS, O1), lambda b, i: (b, i, 0)),
            pl.BlockSpec((1, ROWS, O1), lambda b, i: (b, i, 0)),
        ],
        out_shape=[
            jax.ShapeDtypeStruct((B, N, K), jnp.int32),
            jax.ShapeDtypeStruct((B, N, O1), jnp.float32),
            jax.ShapeDtypeStruct((B, N, O1), jnp.float32),
        ],
        compiler_params=pltpu.CompilerParams(
            dimension_semantics=("parallel", "parallel")),
    )(x, x, w1at, wdt)

    idx_flat = jnp.transpose(idxg.reshape(B * N, K)).reshape(-1)
    gathered = _sc_gather(a_rows.reshape(B * N, O1), idx_flat)

    out = pl.pallas_call(
        _mlp_body,
        grid=(B * N // R3, K),
        in_specs=[
            pl.BlockSpec((1, R3, O1), lambda i, j: (j, i, 0)),
            pl.BlockSpec((R3, O1), lambda i, j: (i, 0)),
            pl.BlockSpec((O1, O2), lambda i, j: (0, 0)),
        ],
        out_specs=pl.BlockSpec((R3, O2), lambda i, j: (i, 0)),
        out_shape=jax.ShapeDtypeStruct((B * N, O2), jnp.float32),
        compiler_params=pltpu.CompilerParams(
            dimension_semantics=("parallel", "arbitrary")),
    )(gathered.reshape(K, B * N, O1), b_rows.reshape(B * N, O1), w2t)

    return jnp.swapaxes(out.reshape(B, N, O2), 1, 2)


# speedup vs baseline: 1.7211x; 1.7211x over previous
import functools

import jax
import jax.numpy as jnp
from jax import lax
from jax.experimental import pallas as pl
from jax.experimental.pallas import tpu as pltpu
from jax.experimental.pallas import tpu_sc as plsc

K = 20
ROWS = 256
R3 = 256
GW = 128


def _knn_body(xfull_ref, xtile_ref, w1at_ref, wdt_ref, idx_ref, a_ref, bv_ref):
    xb = xfull_ref[0]
    xi = xtile_ref[0]
    n = xb.shape[1]
    rows = xi.shape[1]

    dn = (((0,), (0,)), ((), ()))
    a_ref[0] = lax.dot_general(xi, w1at_ref[...], dn,
                               preferred_element_type=jnp.float32)
    bv_ref[0] = lax.dot_general(xi, wdt_ref[...], dn,
                                preferred_element_type=jnp.float32)

    inner = lax.dot_general(xi, xb, dn, preferred_element_type=jnp.float32)
    xsq = jnp.sum(xb * xb, axis=0)
    xsq_i = jnp.sum(xi * xi, axis=0)
    d = 2.0 * inner - xsq[None, :] - xsq_i[:, None]

    nbits = (n - 1).bit_length()
    lowmask = jnp.int32((1 << nbits) - 1)
    rowmin = jnp.min(d, axis=1, keepdims=True)
    scale = (2.0 ** 20 - 2.0) / jnp.maximum(-rowmin, 1e-30)
    qd = lax.convert_element_type(d * scale, jnp.int32)
    cols = lax.broadcasted_iota(jnp.int32, (rows, n), 1)
    key = (qd * jnp.int32(1 << nbits)) | (jnp.int32(n - 1) - cols)

    neg_inf = jnp.int32(-(2 ** 31))
    base = pl.program_id(0) * n
    picks = []
    for _ in range(K):
        m = jnp.max(key, axis=1, keepdims=True)
        picks.append(jnp.int32(n - 1) - (m & lowmask) + base)
        key = jnp.where(key == m, neg_inf, key)
    idx_ref[0] = jnp.concatenate(picks, axis=1)


def _mlp_body(g_ref, bv_ref, w2t_ref, out_ref):
    bv = bv_ref[...]
    w2t = w2t_ref[...]
    acc = None
    for j in range(K):
        h = g_ref[j] + bv
        h = jnp.where(h > 0, h, 0.2 * h)
        h2 = jnp.dot(h, w2t, preferred_element_type=jnp.float32)
        h2 = jnp.where(h2 > 0, h2, 0.2 * h2)
        acc = h2 if acc is None else jnp.maximum(acc, h2)
    out_ref[...] = acc


def _sc_gather(table, idx):
    num = idx.shape[0]
    d_dim = table.shape[1]
    idx2 = idx.reshape(1, num)
    mesh = plsc.VectorSubcoreMesh(core_axis_name="c", subcore_axis_name="s")

    @functools.partial(
        pl.kernel,
        out_type=jax.ShapeDtypeStruct((num, d_dim), table.dtype),
        mesh=mesh,
    )
    def gk(table_hbm, idx_hbm, out_hbm):
        def body(i_vmem, o_vmem):
            pltpu.sync_copy(table_hbm.at[i_vmem.at[0]], o_vmem)

        pltpu.emit_pipeline(
            body,
            grid=(num // GW,),
            in_specs=[pl.BlockSpec((1, GW), lambda i: (0, i))],
            out_specs=[pl.BlockSpec((GW, d_dim), lambda i: (i, 0))],
            core_axis_name=("c", "s"),
            dimension_semantics=(pltpu.PARALLEL,),
        )(idx_hbm, out_hbm)

    return gk(table, idx2)


def kernel(x, W1, W2):
    B, C, N = x.shape
    O1 = W1.shape[0]
    O2 = W2.shape[0]
    w1at = jnp.transpose(W1[:, :C])
    wdt = jnp.transpose(W1[:, C:] - W1[:, :C])
    w2t = jnp.transpose(W2)

    idxg, a_rows, b_rows = pl.pallas_call(
        _knn_body,
        grid=(B, N // ROWS),
        in_specs=[
            pl.BlockSpec((1, C, N), lambda b, i: (b, 0, 0)),
            pl.BlockSpec((1, C, ROWS), lambda b, i: (b, 0, i)),
            pl.BlockSpec((C, O1), lambda b, i: (0, 0)),
            pl.BlockSpec((C, O1), lambda b, i: (0, 0)),
        ],
        out_specs=[
            pl.BlockSpec((1, ROWS, K), lambda b, i: (b, i, 0)),
            pl.BlockSpec((1, ROWS, O1), lambda b, i: (b, i, 0)),
            pl.BlockSpec((1, ROWS, O1), lambda b, i: (b, i, 0)),
        ],
        out_shape=[
            jax.ShapeDtypeStruct((B, N, K), jnp.int32),
            jax.ShapeDtypeStruct((B, N, O1), jnp.float32),
            jax.ShapeDtypeStruct((B, N, O1), jnp.float32),
        ],
        compiler_params=pltpu.CompilerParams(
            dimension_semantics=("parallel", "parallel")),
    )(x, x, w1at, wdt)

    idx_flat = jnp.transpose(idxg.reshape(B * N, K)).reshape(-1)
    gathered = _sc_gather(a_rows.reshape(B * N, O1), idx_flat)

    out = pl.pallas_call(
        _mlp_body,
        grid=(B * N // R3,),
        in_specs=[
            pl.BlockSpec((K, R3, O1), lambda i: (0, i, 0)),
            pl.BlockSpec((R3, O1), lambda i: (i, 0)),
            pl.BlockSpec((O1, O2), lambda i: (0, 0)),
        ],
        out_specs=pl.BlockSpec((R3, O2), lambda i: (i, 0)),
        out_shape=jax.ShapeDtypeStruct((B * N, O2), jnp.float32),
        compiler_params=pltpu.CompilerParams(
            dimension_semantics=("arbitrary",)),
    )(gathered.reshape(K, B * N, O1), b_rows.reshape(B * N, O1), w2t)

    return jnp.swapaxes(out.reshape(B, N, O2), 1, 2)


# speedup vs baseline: 1.8863x; 1.0959x over previous
import functools

import jax
import jax.numpy as jnp
from jax import lax
from jax.experimental import pallas as pl
from jax.experimental.pallas import tpu as pltpu
from jax.experimental.pallas import tpu_sc as plsc

K = 20
ROWS = 256
R3 = 256
GW = 128


def _knn_body(xfull_ref, xtile_ref, w1at_ref, wdt_ref, idx_ref, a_ref, bv_ref):
    xb = xfull_ref[0]
    xi = xtile_ref[0]
    n = xb.shape[1]
    rows = xi.shape[1]

    dn = (((0,), (0,)), ((), ()))
    a_ref[0] = lax.dot_general(xi, w1at_ref[...], dn,
                               preferred_element_type=jnp.float32)
    bv_ref[0] = lax.dot_general(xi, wdt_ref[...], dn,
                                preferred_element_type=jnp.float32)

    inner = lax.dot_general(xi, xb, dn, preferred_element_type=jnp.float32)
    xsq = jnp.sum(xb * xb, axis=0)
    xsq_i = jnp.sum(xi * xi, axis=0)
    d = 2.0 * inner - xsq[None, :] - xsq_i[:, None]

    nbits = (n - 1).bit_length()
    lowmask = jnp.int32((1 << nbits) - 1)
    rowmin = jnp.min(d, axis=1, keepdims=True)
    scale = (2.0 ** 20 - 2.0) / jnp.maximum(-rowmin, 1e-30)
    qd = lax.convert_element_type(d * scale, jnp.int32)
    cols = lax.broadcasted_iota(jnp.int32, (rows, n), 1)
    key = (qd * jnp.int32(1 << nbits)) | (jnp.int32(n - 1) - cols)

    neg_inf = jnp.int32(-(2 ** 31))
    base = pl.program_id(0) * n
    picks = []
    for _ in range(K):
        m = jnp.max(key, axis=1, keepdims=True)
        picks.append(jnp.int32(n - 1) - (m & lowmask) + base)
        key = jnp.where(key == m, neg_inf, key)
    idx_ref[0] = jnp.concatenate(picks, axis=1)


def _mlp_body(g_ref, bv_ref, w2t_ref, out_ref):
    bv = bv_ref[...]
    w2t = w2t_ref[...]
    acc = None
    for j in range(K):
        h = g_ref[j] + bv
        h = jnp.where(h > 0, h, 0.2 * h)
        h2 = jnp.dot(h, w2t, preferred_element_type=jnp.float32)
        h2 = jnp.where(h2 > 0, h2, 0.2 * h2)
        acc = h2 if acc is None else jnp.maximum(acc, h2)
    out_ref[...] = acc


def _sc_gather(table, idx):
    num = idx.shape[0]
    d_dim = table.shape[1]
    idx2 = idx.reshape(1, num)
    mesh = plsc.VectorSubcoreMesh(core_axis_name="c", subcore_axis_name="s")

    @functools.partial(
        pl.kernel,
        out_type=jax.ShapeDtypeStruct((num, d_dim), table.dtype),
        mesh=mesh,
    )
    def gk(table_hbm, idx_hbm, out_hbm):
        def body(i_vmem, o_vmem):
            pltpu.sync_copy(table_hbm.at[i_vmem.at[0]], o_vmem)

        pltpu.emit_pipeline(
            body,
            grid=(num // GW,),
            in_specs=[pl.BlockSpec((1, GW), lambda i: (0, i))],
            out_specs=[pl.BlockSpec((GW, d_dim), lambda i: (i, 0))],
            core_axis_name=("c", "s"),
            dimension_semantics=(pltpu.PARALLEL,),
        )(idx_hbm, out_hbm)

    return gk(table, idx2)


def kernel(x, W1, W2):
    B, C, N = x.shape
    O1 = W1.shape[0]
    O2 = W2.shape[0]
    w1at = jnp.transpose(W1[:, :C])
    wdt = jnp.transpose(W1[:, C:] - W1[:, :C])
    w2t = jnp.transpose(W2)

    outs = []
    for b in range(B):
        xb = lax.slice_in_dim(x, b, b + 1, axis=0)
        idxg, a_rows, b_rows = pl.pallas_call(
            _knn_body,
            grid=(1, N // ROWS),
            in_specs=[
                pl.BlockSpec((1, C, N), lambda bb, i: (bb, 0, 0)),
                pl.BlockSpec((1, C, ROWS), lambda bb, i: (bb, 0, i)),
                pl.BlockSpec((C, O1), lambda bb, i: (0, 0)),
                pl.BlockSpec((C, O1), lambda bb, i: (0, 0)),
            ],
            out_specs=[
                pl.BlockSpec((1, ROWS, K), lambda bb, i: (bb, i, 0)),
                pl.BlockSpec((1, ROWS, O1), lambda bb, i: (bb, i, 0)),
                pl.BlockSpec((1, ROWS, O1), lambda bb, i: (bb, i, 0)),
            ],
            out_shape=[
                jax.ShapeDtypeStruct((1, N, K), jnp.int32),
                jax.ShapeDtypeStruct((1, N, O1), jnp.float32),
                jax.ShapeDtypeStruct((1, N, O1), jnp.float32),
            ],
        )(xb, xb, w1at, wdt)

        idx_flat = jnp.transpose(idxg.reshape(N, K)).reshape(-1)
        gathered = _sc_gather(a_rows.reshape(N, O1), idx_flat)

        out_b = pl.pallas_call(
            _mlp_body,
            grid=(N // R3,),
            in_specs=[
                pl.BlockSpec((K, R3, O1), lambda i: (0, i, 0)),
                pl.BlockSpec((R3, O1), lambda i: (i, 0)),
                pl.BlockSpec((O1, O2), lambda i: (0, 0)),
            ],
            out_specs=pl.BlockSpec((R3, O2), lambda i: (i, 0)),
            out_shape=jax.ShapeDtypeStruct((N, O2), jnp.float32),
        )(gathered.reshape(K, N, O1), b_rows.reshape(N, O1), w2t)
        outs.append(out_b)

    return jnp.swapaxes(jnp.stack(outs), 1, 2)


# speedup vs baseline: 2.3703x; 1.2566x over previous
import functools

import jax
import jax.numpy as jnp
from jax import lax
from jax.experimental import pallas as pl
from jax.experimental.pallas import tpu as pltpu
from jax.experimental.pallas import tpu_sc as plsc

K = 20
ROWS = 256
R3 = 256
GW = 128


def _batcher_pairs(n):
    pairs = []

    def merge(lo, m, r):
        step = r * 2
        if step < m:
            merge(lo, m, step)
            merge(lo + r, m, step)
            for i in range(lo + r, lo + m - r, step):
                pairs.append((i, i + r))
        else:
            pairs.append((lo, lo + r))

    def sort(lo, m):
        if m > 1:
            mid = m // 2
            sort(lo, mid)
            sort(lo + mid, mid)
            merge(lo, m, 1)

    sort(0, n)
    return pairs


def _knn_body(xfull_ref, xtile_ref, w1at_ref, wdt_ref, idx_ref, a_ref, bv_ref):
    xb = xfull_ref[0]
    xi = xtile_ref[0]
    n = xb.shape[1]
    rows = xi.shape[1]

    dn = (((0,), (0,)), ((), ()))
    a_ref[0] = lax.dot_general(xi, w1at_ref[...], dn,
                               preferred_element_type=jnp.float32)
    bv_ref[0] = lax.dot_general(xi, wdt_ref[...], dn,
                                preferred_element_type=jnp.float32)

    inner = lax.dot_general(xi, xb, dn, preferred_element_type=jnp.float32)
    xsq = jnp.sum(xb * xb, axis=0)
    xsq_i = jnp.sum(xi * xi, axis=0)
    d = 2.0 * inner - xsq[None, :] - xsq_i[:, None]

    nbits = (n - 1).bit_length()
    lowmask = jnp.int32((1 << nbits) - 1)
    rowmin = jnp.min(d, axis=1, keepdims=True)
    scale = (2.0 ** 20 - 2.0) / jnp.maximum(-rowmin, 1e-30)
    qd = lax.convert_element_type(d * scale, jnp.int32)
    cols = lax.broadcasted_iota(jnp.int32, (rows, n), 1)
    key = (qd * jnp.int32(1 << nbits)) | (jnp.int32(n - 1) - cols)

    nlay = n // 128
    layers = [key[:, c * 128:(c + 1) * 128] for c in range(nlay)]
    for i, j in _batcher_pairs(nlay):
        a, b = layers[i], layers[j]
        layers[i] = jnp.maximum(a, b)
        layers[j] = jnp.minimum(a, b)

    neg_inf = jnp.int32(-(2 ** 31))
    base = pl.program_id(0) * n
    picks = []
    for t in range(K):
        head = layers[0]
        m = jnp.max(head, axis=1, keepdims=True)
        picks.append(jnp.int32(n - 1) - (m & lowmask) + base)
        if t == K - 1:
            break
        hit = head == m
        depth = min(nlay - 1, K - 1 - t)
        for i in range(depth):
            layers[i] = jnp.where(hit, layers[i + 1], layers[i])
        if depth == nlay - 1:
            layers[depth] = jnp.where(hit, neg_inf, layers[depth])
    idx_ref[0] = jnp.concatenate(picks, axis=1)


def _mlp_body(g_ref, bv_ref, w2t_ref, out_ref):
    bv = bv_ref[...]
    w2t = w2t_ref[...]
    acc = None
    for j in range(K):
        h = g_ref[j] + bv
        h = jnp.where(h > 0, h, 0.2 * h)
        h2 = jnp.dot(h, w2t, preferred_element_type=jnp.float32)
        h2 = jnp.where(h2 > 0, h2, 0.2 * h2)
        acc = h2 if acc is None else jnp.maximum(acc, h2)
    out_ref[...] = acc


def _sc_gather(table, idx):
    num = idx.shape[0]
    d_dim = table.shape[1]
    idx2 = idx.reshape(1, num)
    mesh = plsc.VectorSubcoreMesh(core_axis_name="c", subcore_axis_name="s")

    @functools.partial(
        pl.kernel,
        out_type=jax.ShapeDtypeStruct((num, d_dim), table.dtype),
        mesh=mesh,
    )
    def gk(table_hbm, idx_hbm, out_hbm):
        def body(i_vmem, o_vmem):
            pltpu.sync_copy(table_hbm.at[i_vmem.at[0]], o_vmem)

        pltpu.emit_pipeline(
            body,
            grid=(num // GW,),
            in_specs=[pl.BlockSpec((1, GW), lambda i: (0, i))],
            out_specs=[pl.BlockSpec((GW, d_dim), lambda i: (i, 0))],
            core_axis_name=("c", "s"),
            dimension_semantics=(pltpu.PARALLEL,),
        )(idx_hbm, out_hbm)

    return gk(table, idx2)


def kernel(x, W1, W2):
    B, C, N = x.shape
    O1 = W1.shape[0]
    O2 = W2.shape[0]
    w1at = jnp.transpose(W1[:, :C])
    wdt = jnp.transpose(W1[:, C:] - W1[:, :C])
    w2t = jnp.transpose(W2)

    outs = []
    for b in range(B):
        xb = lax.slice_in_dim(x, b, b + 1, axis=0)
        idxg, a_rows, b_rows = pl.pallas_call(
            _knn_body,
            grid=(1, N // ROWS),
            in_specs=[
                pl.BlockSpec((1, C, N), lambda bb, i: (bb, 0, 0)),
                pl.BlockSpec((1, C, ROWS), lambda bb, i: (bb, 0, i)),
                pl.BlockSpec((C, O1), lambda bb, i: (0, 0)),
                pl.BlockSpec((C, O1), lambda bb, i: (0, 0)),
            ],
            out_specs=[
                pl.BlockSpec((1, ROWS, K), lambda bb, i: (bb, i, 0)),
                pl.BlockSpec((1, ROWS, O1), lambda bb, i: (bb, i, 0)),
                pl.BlockSpec((1, ROWS, O1), lambda bb, i: (bb, i, 0)),
            ],
            out_shape=[
                jax.ShapeDtypeStruct((1, N, K), jnp.int32),
                jax.ShapeDtypeStruct((1, N, O1), jnp.float32),
                jax.ShapeDtypeStruct((1, N, O1), jnp.float32),
            ],
        )(xb, xb, w1at, wdt)

        idx_flat = jnp.transpose(idxg.reshape(N, K)).reshape(-1)
        gathered = _sc_gather(a_rows.reshape(N, O1), idx_flat)

        out_b = pl.pallas_call(
            _mlp_body,
            grid=(N // R3,),
            in_specs=[
                pl.BlockSpec((K, R3, O1), lambda i: (0, i, 0)),
                pl.BlockSpec((R3, O1), lambda i: (i, 0)),
                pl.BlockSpec((O1, O2), lambda i: (0, 0)),
            ],
            out_specs=pl.BlockSpec((R3, O2), lambda i: (i, 0)),
            out_shape=jax.ShapeDtypeStruct((N, O2), jnp.float32),
        )(gathered.reshape(K, N, O1), b_rows.reshape(N, O1), w2t)
        outs.append(out_b)

    return jnp.swapaxes(jnp.stack(outs), 1, 2)


# speedup vs baseline: 2.5619x; 1.0808x over previous
import functools

import jax
import jax.numpy as jnp
from jax import lax
from jax.experimental import pallas as pl
from jax.experimental.pallas import tpu as pltpu
from jax.experimental.pallas import tpu_sc as plsc

K = 20
ROWS = 256
R3 = 256
GW = 128


def _batcher_pairs(n):
    pairs = []

    def merge(lo, m, r):
        step = r * 2
        if step < m:
            merge(lo, m, step)
            merge(lo + r, m, step)
            for i in range(lo + r, lo + m - r, step):
                pairs.append((i, i + r))
        else:
            pairs.append((lo, lo + r))

    def sort(lo, m):
        if m > 1:
            mid = m // 2
            sort(lo, mid)
            sort(lo + mid, mid)
            merge(lo, m, 1)

    sort(0, n)
    return pairs


def _knn_body(xfull_ref, xtile_ref, w1at_ref, wdt_ref, idx_ref, a_ref, bv_ref):
    xb = xfull_ref[0]
    xi = xtile_ref[0]
    n = xb.shape[1]
    rows = xi.shape[1]

    dn = (((0,), (0,)), ((), ()))
    a_ref[0] = lax.dot_general(xi, w1at_ref[...], dn,
                               preferred_element_type=jnp.float32)
    bv_ref[0] = lax.dot_general(xi, wdt_ref[...], dn,
                                preferred_element_type=jnp.float32)

    inner = lax.dot_general(xi, xb, dn, preferred_element_type=jnp.float32)
    xsq = jnp.sum(xb * xb, axis=0)
    xsq_i = jnp.sum(xi * xi, axis=0)
    d = 2.0 * inner - xsq[None, :] - xsq_i[:, None]

    nbits = (n - 1).bit_length()
    lowmask = jnp.int32((1 << nbits) - 1)
    offset = jnp.int32(2 ** 20 - 4099)
    rowmin = jnp.min(d, axis=1, keepdims=True)
    scale = (2.0 ** 20 - 4100.0) / jnp.maximum(-rowmin, 1e-30)
    qd = jnp.minimum(lax.convert_element_type(d * scale, jnp.int32), 0)
    cols = lax.broadcasted_iota(jnp.int32, (rows, n), 1)
    key_i = ((qd + offset) * jnp.int32(1 << nbits)) | (jnp.int32(n - 1) - cols)
    key = lax.bitcast_convert_type(key_i, jnp.float32)

    nlay = n // 128
    layers = [key[:, c * 128:(c + 1) * 128] for c in range(nlay)]
    for i, j in _batcher_pairs(nlay):
        a, b = layers[i], layers[j]
        layers[i] = jnp.maximum(a, b)
        layers[j] = jnp.minimum(a, b)

    sentinel = jnp.float32(0.0)
    base = pl.program_id(0) * n
    picks = []
    for t in range(K):
        head = layers[0]
        m = jnp.max(head, axis=1, keepdims=True)
        m_i = lax.bitcast_convert_type(m, jnp.int32)
        picks.append(jnp.int32(n - 1) - (m_i & lowmask) + base)
        if t == K - 1:
            break
        hit = head == m
        depth = min(nlay - 1, K - 1 - t)
        for i in range(depth):
            layers[i] = jnp.where(hit, layers[i + 1], layers[i])
        if depth == nlay - 1:
            layers[depth] = jnp.where(hit, sentinel, layers[depth])
    idx_ref[0] = jnp.concatenate(picks, axis=1)


def _mlp_body(g_ref, bv_ref, w2t_ref, out_ref):
    bv = bv_ref[...]
    w2t = w2t_ref[...]
    acc = None
    for j in range(K):
        h = g_ref[j] + bv
        h = jnp.where(h > 0, h, 0.2 * h)
        h2 = jnp.dot(h, w2t, preferred_element_type=jnp.float32)
        h2 = jnp.where(h2 > 0, h2, 0.2 * h2)
        acc = h2 if acc is None else jnp.maximum(acc, h2)
    out_ref[...] = acc


def _sc_gather(table, idx):
    num = idx.shape[0]
    d_dim = table.shape[1]
    idx2 = idx.reshape(1, num)
    mesh = plsc.VectorSubcoreMesh(core_axis_name="c", subcore_axis_name="s")

    @functools.partial(
        pl.kernel,
        out_type=jax.ShapeDtypeStruct((num, d_dim), table.dtype),
        mesh=mesh,
    )
    def gk(table_hbm, idx_hbm, out_hbm):
        def body(i_vmem, o_vmem):
            pltpu.sync_copy(table_hbm.at[i_vmem.at[0]], o_vmem)

        pltpu.emit_pipeline(
            body,
            grid=(num // GW,),
            in_specs=[pl.BlockSpec((1, GW), lambda i: (0, i))],
            out_specs=[pl.BlockSpec((GW, d_dim), lambda i: (i, 0))],
            core_axis_name=("c", "s"),
            dimension_semantics=(pltpu.PARALLEL,),
        )(idx_hbm, out_hbm)

    return gk(table, idx2)


def kernel(x, W1, W2):
    B, C, N = x.shape
    O1 = W1.shape[0]
    O2 = W2.shape[0]
    w1at = jnp.transpose(W1[:, :C])
    wdt = jnp.transpose(W1[:, C:] - W1[:, :C])
    w2t = jnp.transpose(W2)

    outs = []
    for b in range(B):
        xb = lax.slice_in_dim(x, b, b + 1, axis=0)
        idxg, a_rows, b_rows = pl.pallas_call(
            _knn_body,
            grid=(1, N // ROWS),
            in_specs=[
                pl.BlockSpec((1, C, N), lambda bb, i: (bb, 0, 0)),
                pl.BlockSpec((1, C, ROWS), lambda bb, i: (bb, 0, i)),
                pl.BlockSpec((C, O1), lambda bb, i: (0, 0)),
                pl.BlockSpec((C, O1), lambda bb, i: (0, 0)),
            ],
            out_specs=[
                pl.BlockSpec((1, ROWS, K), lambda bb, i: (bb, i, 0)),
                pl.BlockSpec((1, ROWS, O1), lambda bb, i: (bb, i, 0)),
                pl.BlockSpec((1, ROWS, O1), lambda bb, i: (bb, i, 0)),
            ],
            out_shape=[
                jax.ShapeDtypeStruct((1, N, K), jnp.int32),
                jax.ShapeDtypeStruct((1, N, O1), jnp.float32),
                jax.ShapeDtypeStruct((1, N, O1), jnp.float32),
            ],
        )(xb, xb, w1at, wdt)

        idx_flat = jnp.transpose(idxg.reshape(N, K)).reshape(-1)
        gathered = _sc_gather(a_rows.reshape(N, O1), idx_flat)

        out_b = pl.pallas_call(
            _mlp_body,
            grid=(N // R3,),
            in_specs=[
                pl.BlockSpec((K, R3, O1), lambda i: (0, i, 0)),
                pl.BlockSpec((R3, O1), lambda i: (i, 0)),
                pl.BlockSpec((O1, O2), lambda i: (0, 0)),
            ],
            out_specs=pl.BlockSpec((R3, O2), lambda i: (i, 0)),
            out_shape=jax.ShapeDtypeStruct((N, O2), jnp.float32),
        )(gathered.reshape(K, N, O1), b_rows.reshape(N, O1), w2t)
        outs.append(out_b)

    return jnp.swapaxes(jnp.stack(outs), 1, 2)


# speedup vs baseline: 2.5664x; 1.0018x over previous
import functools

import jax
import jax.numpy as jnp
from jax import lax
from jax.experimental import pallas as pl
from jax.experimental.pallas import tpu as pltpu
from jax.experimental.pallas import tpu_sc as plsc

K = 20
ROWS = 256
R3 = 256
GW = 128


def _batcher_pairs(n):
    pairs = []

    def merge(lo, m, r):
        step = r * 2
        if step < m:
            merge(lo, m, step)
            merge(lo + r, m, step)
            for i in range(lo + r, lo + m - r, step):
                pairs.append((i, i + r))
        else:
            pairs.append((lo, lo + r))

    def sort(lo, m):
        if m > 1:
            mid = m // 2
            sort(lo, mid)
            sort(lo + mid, mid)
            merge(lo, m, 1)

    sort(0, n)
    return pairs


def _knn_body(xfull_ref, xtile_ref, w1at_ref, wdt_ref, idx_ref, a_ref, bv_ref):
    xb = xfull_ref[0]
    xi = xtile_ref[0]
    n = xb.shape[1]
    rows = xi.shape[1]

    dn = (((0,), (0,)), ((), ()))
    a_ref[0] = lax.dot_general(xi, w1at_ref[...], dn,
                               preferred_element_type=jnp.float32)
    bv_ref[0] = lax.dot_general(xi, wdt_ref[...], dn,
                                preferred_element_type=jnp.float32)

    inner = lax.dot_general(xi, xb, dn, preferred_element_type=jnp.float32)
    xsq = jnp.sum(xb * xb, axis=0)
    xsq_i = jnp.sum(xi * xi, axis=0)
    d = 2.0 * inner - xsq[None, :] - xsq_i[:, None]

    nbits = (n - 1).bit_length()
    lowmask = jnp.int32((1 << nbits) - 1)
    offset = jnp.int32(2 ** 20 - 4099)
    rowmin = jnp.min(d, axis=1, keepdims=True)
    scale = (2.0 ** 20 - 4100.0) / jnp.maximum(-rowmin, 1e-30)
    qd = jnp.minimum(lax.convert_element_type(d * scale, jnp.int32), 0)
    cols = lax.broadcasted_iota(jnp.int32, (rows, n), 1)
    key_i = ((qd + offset) * jnp.int32(1 << nbits)) | (jnp.int32(n - 1) - cols)
    key = lax.bitcast_convert_type(key_i, jnp.float32)

    nlay = n // 128
    layers = [key[:, c * 128:(c + 1) * 128] for c in range(nlay)]
    for i, j in _batcher_pairs(nlay):
        a, b = layers[i], layers[j]
        layers[i] = jnp.maximum(a, b)
        layers[j] = jnp.minimum(a, b)

    sentinel = jnp.float32(0.0)
    base = pl.program_id(0) * n
    picks = []
    for t in range(K):
        head = layers[0]
        m = jnp.max(head, axis=1, keepdims=True)
        m_i = lax.bitcast_convert_type(m, jnp.int32)
        picks.append(jnp.int32(n - 1) - (m_i & lowmask) + base)
        if t == K - 1:
            break
        hit = head == m
        depth = min(nlay - 1, K - 1 - t)
        for i in range(depth):
            layers[i] = jnp.where(hit, layers[i + 1], layers[i])
        if depth == nlay - 1:
            layers[depth] = jnp.where(hit, sentinel, layers[depth])
    idx_ref[...] = jnp.transpose(jnp.concatenate(picks, axis=1))


def _mlp_body(g_ref, bv_ref, w2t_ref, out_ref):
    bv = bv_ref[...]
    w2t = w2t_ref[...]
    acc = None
    for j in range(K):
        h = g_ref[j] + bv
        h = jnp.where(h > 0, h, 0.2 * h)
        h2 = jnp.dot(h, w2t, preferred_element_type=jnp.float32)
        h2 = jnp.where(h2 > 0, h2, 0.2 * h2)
        acc = h2 if acc is None else jnp.maximum(acc, h2)
    out_ref[...] = acc


def _sc_gather(table, idx):
    num = idx.shape[0]
    d_dim = table.shape[1]
    idx2 = idx.reshape(1, num)
    mesh = plsc.VectorSubcoreMesh(core_axis_name="c", subcore_axis_name="s")

    @functools.partial(
        pl.kernel,
        out_type=jax.ShapeDtypeStruct((num, d_dim), table.dtype),
        mesh=mesh,
    )
    def gk(table_hbm, idx_hbm, out_hbm):
        def body(i_vmem, o_vmem):
            pltpu.sync_copy(table_hbm.at[i_vmem.at[0]], o_vmem)

        pltpu.emit_pipeline(
            body,
            grid=(num // GW,),
            in_specs=[pl.BlockSpec((1, GW), lambda i: (0, i))],
            out_specs=[pl.BlockSpec((GW, d_dim), lambda i: (i, 0))],
            core_axis_name=("c", "s"),
            dimension_semantics=(pltpu.PARALLEL,),
        )(idx_hbm, out_hbm)

    return gk(table, idx2)


def kernel(x, W1, W2):
    B, C, N = x.shape
    O1 = W1.shape[0]
    O2 = W2.shape[0]
    w1at = jnp.transpose(W1[:, :C])
    wdt = jnp.transpose(W1[:, C:] - W1[:, :C])
    w2t = jnp.transpose(W2)

    outs = []
    for b in range(B):
        xb = lax.slice_in_dim(x, b, b + 1, axis=0)
        idxg, a_rows, b_rows = pl.pallas_call(
            _knn_body,
            grid=(1, N // ROWS),
            in_specs=[
                pl.BlockSpec((1, C, N), lambda bb, i: (bb, 0, 0)),
                pl.BlockSpec((1, C, ROWS), lambda bb, i: (bb, 0, i)),
                pl.BlockSpec((C, O1), lambda bb, i: (0, 0)),
                pl.BlockSpec((C, O1), lambda bb, i: (0, 0)),
            ],
            out_specs=[
                pl.BlockSpec((K, ROWS), lambda bb, i: (0, i)),
                pl.BlockSpec((1, ROWS, O1), lambda bb, i: (bb, i, 0)),
                pl.BlockSpec((1, ROWS, O1), lambda bb, i: (bb, i, 0)),
            ],
            out_shape=[
                jax.ShapeDtypeStruct((K, N), jnp.int32),
                jax.ShapeDtypeStruct((1, N, O1), jnp.float32),
                jax.ShapeDtypeStruct((1, N, O1), jnp.float32),
            ],
        )(xb, xb, w1at, wdt)

        gathered = _sc_gather(a_rows.reshape(N, O1), idxg.reshape(K * N))

        out_b = pl.pallas_call(
            _mlp_body,
            grid=(N // R3,),
            in_specs=[
                pl.BlockSpec((K, R3, O1), lambda i: (0, i, 0)),
                pl.BlockSpec((R3, O1), lambda i: (i, 0)),
                pl.BlockSpec((O1, O2), lambda i: (0, 0)),
            ],
            out_specs=pl.BlockSpec((R3, O2), lambda i: (i, 0)),
            out_shape=jax.ShapeDtypeStruct((N, O2), jnp.float32),
        )(gathered.reshape(K, N, O1), b_rows.reshape(N, O1), w2t)
        outs.append(out_b)

    return jnp.swapaxes(jnp.stack(outs), 1, 2)


# speedup vs baseline: 2.6274x; 1.0238x over previous
import functools

import jax
import jax.numpy as jnp
from jax import lax
from jax.experimental import pallas as pl
from jax.experimental.pallas import tpu as pltpu
from jax.experimental.pallas import tpu_sc as plsc

K = 20
ROWS = 256
R3 = 512
GW = 128


def _batcher_pairs(n):
    pairs = []

    def merge(lo, m, r):
        step = r * 2
        if step < m:
            merge(lo, m, step)
            merge(lo + r, m, step)
            for i in range(lo + r, lo + m - r, step):
                pairs.append((i, i + r))
        else:
            pairs.append((lo, lo + r))

    def sort(lo, m):
        if m > 1:
            mid = m // 2
            sort(lo, mid)
            sort(lo + mid, mid)
            merge(lo, m, 1)

    sort(0, n)
    return pairs


def _knn_body(xfull_ref, xtile_ref, w1at_ref, wdt_ref, idx_ref, a_ref, bv_ref):
    xb = xfull_ref[0]
    xi = xtile_ref[0]
    n = xb.shape[1]
    rows = xi.shape[1]

    dn = (((0,), (0,)), ((), ()))
    a_ref[0] = lax.dot_general(xi, w1at_ref[...], dn,
                               preferred_element_type=jnp.float32)
    bv_ref[0] = lax.dot_general(xi, wdt_ref[...], dn,
                                preferred_element_type=jnp.float32)

    inner = lax.dot_general(xi, xb, dn, preferred_element_type=jnp.float32)
    xsq = jnp.sum(xb * xb, axis=0)
    xsq_i = jnp.sum(xi * xi, axis=0)
    d = 2.0 * inner - xsq[None, :] - xsq_i[:, None]

    nbits = (n - 1).bit_length()
    lowmask = jnp.int32((1 << nbits) - 1)
    offset = jnp.int32(2 ** 20 - 4099)
    rowmin = jnp.min(d, axis=1, keepdims=True)
    scale = (2.0 ** 20 - 4100.0) / jnp.maximum(-rowmin, 1e-30)
    qd = jnp.minimum(lax.convert_element_type(d * scale, jnp.int32), 0)
    cols = lax.broadcasted_iota(jnp.int32, (rows, n), 1)
    key_i = ((qd + offset) * jnp.int32(1 << nbits)) | (jnp.int32(n - 1) - cols)
    key = lax.bitcast_convert_type(key_i, jnp.float32)

    nlay = n // 128
    layers = [key[:, c * 128:(c + 1) * 128] for c in range(nlay)]
    for i, j in _batcher_pairs(nlay):
        a, b = layers[i], layers[j]
        layers[i] = jnp.maximum(a, b)
        layers[j] = jnp.minimum(a, b)

    sentinel = jnp.float32(0.0)
    base = pl.program_id(0) * n
    picks = []
    for t in range(K):
        head = layers[0]
        m = jnp.max(head, axis=1, keepdims=True)
        m_i = lax.bitcast_convert_type(m, jnp.int32)
        picks.append(jnp.int32(n - 1) - (m_i & lowmask) + base)
        if t == K - 1:
            break
        hit = head == m
        depth = min(nlay - 1, K - 1 - t)
        for i in range(depth):
            layers[i] = jnp.where(hit, layers[i + 1], layers[i])
        if depth == nlay - 1:
            layers[depth] = jnp.where(hit, sentinel, layers[depth])
    idx_ref[...] = jnp.transpose(jnp.concatenate(picks, axis=1))


def _mlp_body(g_ref, bv_ref, w2_ref, out_ref):
    bv = bv_ref[...]
    w2 = w2_ref[...]
    dn = (((1,), (1,)), ((), ()))
    acc = None
    for j in range(K):
        h = g_ref[j] + bv
        h = jnp.where(h > 0, h, 0.2 * h)
        h2 = lax.dot_general(w2, h, dn, preferred_element_type=jnp.float32)
        h2 = jnp.where(h2 > 0, h2, 0.2 * h2)
        acc = h2 if acc is None else jnp.maximum(acc, h2)
    out_ref[...] = acc


def _sc_gather(table, idx):
    num = idx.shape[0]
    d_dim = table.shape[1]
    idx2 = idx.reshape(1, num)
    mesh = plsc.VectorSubcoreMesh(core_axis_name="c", subcore_axis_name="s")

    @functools.partial(
        pl.kernel,
        out_type=jax.ShapeDtypeStruct((num, d_dim), table.dtype),
        mesh=mesh,
    )
    def gk(table_hbm, idx_hbm, out_hbm):
        def body(i_vmem, o_vmem):
            pltpu.sync_copy(table_hbm.at[i_vmem.at[0]], o_vmem)

        pltpu.emit_pipeline(
            body,
            grid=(num // GW,),
            in_specs=[pl.BlockSpec((1, GW), lambda i: (0, i))],
            out_specs=[pl.BlockSpec((GW, d_dim), lambda i: (i, 0))],
            core_axis_name=("c", "s"),
            dimension_semantics=(pltpu.PARALLEL,),
        )(idx_hbm, out_hbm)

    return gk(table, idx2)


def kernel(x, W1, W2):
    B, C, N = x.shape
    O1 = W1.shape[0]
    O2 = W2.shape[0]
    w1at = jnp.transpose(W1[:, :C])
    wdt = jnp.transpose(W1[:, C:] - W1[:, :C])

    outs = []
    for b in range(B):
        xb = lax.slice_in_dim(x, b, b + 1, axis=0)
        idxg, a_rows, b_rows = pl.pallas_call(
            _knn_body,
            grid=(1, N // ROWS),
            in_specs=[
                pl.BlockSpec((1, C, N), lambda bb, i: (bb, 0, 0)),
                pl.BlockSpec((1, C, ROWS), lambda bb, i: (bb, 0, i)),
                pl.BlockSpec((C, O1), lambda bb, i: (0, 0)),
                pl.BlockSpec((C, O1), lambda bb, i: (0, 0)),
            ],
            out_specs=[
                pl.BlockSpec((K, ROWS), lambda bb, i: (0, i)),
                pl.BlockSpec((1, ROWS, O1), lambda bb, i: (bb, i, 0)),
                pl.BlockSpec((1, ROWS, O1), lambda bb, i: (bb, i, 0)),
            ],
            out_shape=[
                jax.ShapeDtypeStruct((K, N), jnp.int32),
                jax.ShapeDtypeStruct((1, N, O1), jnp.float32),
                jax.ShapeDtypeStruct((1, N, O1), jnp.float32),
            ],
        )(xb, xb, w1at, wdt)

        gathered = _sc_gather(a_rows.reshape(N, O1), idxg.reshape(K * N))

        out_b = pl.pallas_call(
            _mlp_body,
            grid=(N // R3,),
            in_specs=[
                pl.BlockSpec((K, R3, O1), lambda i: (0, i, 0)),
                pl.BlockSpec((R3, O1), lambda i: (i, 0)),
                pl.BlockSpec((O2, O1), lambda i: (0, 0)),
            ],
            out_specs=pl.BlockSpec((O2, R3), lambda i: (0, i)),
            out_shape=jax.ShapeDtypeStruct((O2, N), jnp.float32),
        )(gathered.reshape(K, N, O1), b_rows.reshape(N, O1), W2)
        outs.append(out_b)

    return jnp.stack(outs)


# speedup vs baseline: 2.6409x; 1.0051x over previous
import functools

import jax
import jax.numpy as jnp
from jax import lax
from jax.experimental import pallas as pl
from jax.experimental.pallas import tpu as pltpu
from jax.experimental.pallas import tpu_sc as plsc

K = 20
ROWS = 512
R3 = 512
GW = 128


def _batcher_pairs(n):
    pairs = []

    def merge(lo, m, r):
        step = r * 2
        if step < m:
            merge(lo, m, step)
            merge(lo + r, m, step)
            for i in range(lo + r, lo + m - r, step):
                pairs.append((i, i + r))
        else:
            pairs.append((lo, lo + r))

    def sort(lo, m):
        if m > 1:
            mid = m // 2
            sort(lo, mid)
            sort(lo + mid, mid)
            merge(lo, m, 1)

    sort(0, n)
    return pairs


def _knn_body(xfull_ref, xtile_ref, w1at_ref, wdt_ref, idx_ref, a_ref, bv_ref):
    xb = xfull_ref[0]
    xi = xtile_ref[0]
    n = xb.shape[1]
    rows = xi.shape[1]

    dn = (((0,), (0,)), ((), ()))
    a_ref[0] = lax.dot_general(xi, w1at_ref[...], dn,
                               preferred_element_type=jnp.float32)
    bv_ref[0] = lax.dot_general(xi, wdt_ref[...], dn,
                                preferred_element_type=jnp.float32)

    inner = lax.dot_general(xi, xb, dn, preferred_element_type=jnp.float32)
    xsq = jnp.sum(xb * xb, axis=0)
    xsq_i = jnp.sum(xi * xi, axis=0)
    d = 2.0 * inner - xsq[None, :] - xsq_i[:, None]

    nbits = (n - 1).bit_length()
    lowmask = jnp.int32((1 << nbits) - 1)
    offset = jnp.int32(2 ** 20 - 4099)
    rowmin = jnp.min(d, axis=1, keepdims=True)
    scale = (2.0 ** 20 - 4100.0) / jnp.maximum(-rowmin, 1e-30)
    qd = jnp.minimum(lax.convert_element_type(d * scale, jnp.int32), 0)
    cols = lax.broadcasted_iota(jnp.int32, (rows, n), 1)
    key_i = ((qd + offset) * jnp.int32(1 << nbits)) | (jnp.int32(n - 1) - cols)
    key = lax.bitcast_convert_type(key_i, jnp.float32)

    nlay = n // 128
    layers = [key[:, c * 128:(c + 1) * 128] for c in range(nlay)]
    for i, j in _batcher_pairs(nlay):
        a, b = layers[i], layers[j]
        layers[i] = jnp.maximum(a, b)
        layers[j] = jnp.minimum(a, b)

    sentinel = jnp.float32(0.0)
    base = pl.program_id(0) * n
    picks = []
    for t in range(K):
        head = layers[0]
        m = jnp.max(head, axis=1, keepdims=True)
        m_i = lax.bitcast_convert_type(m, jnp.int32)
        picks.append(jnp.int32(n - 1) - (m_i & lowmask) + base)
        if t == K - 1:
            break
        hit = head == m
        depth = min(nlay - 1, K - 1 - t)
        for i in range(depth):
            layers[i] = jnp.where(hit, layers[i + 1], layers[i])
        if depth == nlay - 1:
            layers[depth] = jnp.where(hit, sentinel, layers[depth])
    idx_ref[...] = jnp.transpose(jnp.concatenate(picks, axis=1))


def _mlp_body(g_ref, bv_ref, w2_ref, out_ref):
    bv = bv_ref[...]
    w2 = w2_ref[...]
    dn = (((1,), (1,)), ((), ()))
    acc = None
    for j in range(K):
        h = g_ref[j] + bv
        h = jnp.where(h > 0, h, 0.2 * h)
        h2 = lax.dot_general(w2, h, dn, preferred_element_type=jnp.float32)
        h2 = jnp.where(h2 > 0, h2, 0.2 * h2)
        acc = h2 if acc is None else jnp.maximum(acc, h2)
    out_ref[...] = acc


def _sc_gather(table, idx):
    num = idx.shape[0]
    d_dim = table.shape[1]
    idx2 = idx.reshape(1, num)
    mesh = plsc.VectorSubcoreMesh(core_axis_name="c", subcore_axis_name="s")

    @functools.partial(
        pl.kernel,
        out_type=jax.ShapeDtypeStruct((num, d_dim), table.dtype),
        mesh=mesh,
    )
    def gk(table_hbm, idx_hbm, out_hbm):
        def body(i_vmem, o_vmem):
            pltpu.sync_copy(table_hbm.at[i_vmem.at[0]], o_vmem)

        pltpu.emit_pipeline(
            body,
            grid=(num // GW,),
            in_specs=[pl.BlockSpec((1, GW), lambda i: (0, i))],
            out_specs=[pl.BlockSpec((GW, d_dim), lambda i: (i, 0))],
            core_axis_name=("c", "s"),
            dimension_semantics=(pltpu.PARALLEL,),
        )(idx_hbm, out_hbm)

    return gk(table, idx2)


def kernel(x, W1, W2):
    B, C, N = x.shape
    O1 = W1.shape[0]
    O2 = W2.shape[0]
    w1at = jnp.transpose(W1[:, :C])
    wdt = jnp.transpose(W1[:, C:] - W1[:, :C])

    outs = []
    for b in range(B):
        xb = lax.slice_in_dim(x, b, b + 1, axis=0)
        idxg, a_rows, b_rows = pl.pallas_call(
            _knn_body,
            grid=(1, N // ROWS),
            in_specs=[
                pl.BlockSpec((1, C, N), lambda bb, i: (bb, 0, 0)),
                pl.BlockSpec((1, C, ROWS), lambda bb, i: (bb, 0, i)),
                pl.BlockSpec((C, O1), lambda bb, i: (0, 0)),
                pl.BlockSpec((C, O1), lambda bb, i: (0, 0)),
            ],
            out_specs=[
                pl.BlockSpec((K, ROWS), lambda bb, i: (0, i)),
                pl.BlockSpec((1, ROWS, O1), lambda bb, i: (bb, i, 0)),
                pl.BlockSpec((1, ROWS, O1), lambda bb, i: (bb, i, 0)),
            ],
            out_shape=[
                jax.ShapeDtypeStruct((K, N), jnp.int32),
                jax.ShapeDtypeStruct((1, N, O1), jnp.float32),
                jax.ShapeDtypeStruct((1, N, O1), jnp.float32),
            ],
        )(xb, xb, w1at, wdt)

        gathered = _sc_gather(a_rows.reshape(N, O1), idxg.reshape(K * N))

        out_b = pl.pallas_call(
            _mlp_body,
            grid=(N // R3,),
            in_specs=[
                pl.BlockSpec((K, R3, O1), lambda i: (0, i, 0)),
                pl.BlockSpec((R3, O1), lambda i: (i, 0)),
                pl.BlockSpec((O2, O1), lambda i: (0, 0)),
            ],
            out_specs=pl.BlockSpec((O2, R3), lambda i: (0, i)),
            out_shape=jax.ShapeDtypeStruct((O2, N), jnp.float32),
        )(gathered.reshape(K, N, O1), b_rows.reshape(N, O1), W2)
        outs.append(out_b)

    return jnp.stack(outs)
